# baseline (speedup 1.0000x reference)
import jax
import jax.numpy as jnp
from jax.experimental import pallas as pl
from jax.experimental.pallas import tpu as pltpu

_B = 4096
_D = 1024
_K = 50
_P = 10
_SIGMA = 0.5
_ALPHA = 0.1

_RB = 256
_C = 256
_NCH = _B // _C
_RBP = 512


def _prep_body(emb_ref, w_ref, z_ref, zw_ref):
    x = emb_ref[...]
    ss = jnp.sum(x * x, axis=1, keepdims=True)
    norm = jnp.maximum(jnp.sqrt(ss), 1e-12)
    z = x / norm
    z_ref[...] = z
    zw_ref[...] = jax.lax.dot_general(
        z, w_ref[...], (((1,), (0,)), ((), ())),
        preferred_element_type=jnp.float32)[:, 0]


def _knn_body(z_row_ref, z_all_ref, knn_ref, buf_ref):
    s = pl.program_id(0)
    smw = jax.lax.rem(s, 2)
    smr = jax.lax.rem(s + 1, 2)
    zr = z_row_ref[...]
    ones_b = jnp.ones((_B, 1), jnp.float32)

    def it(t, carry):
        lo, hi = carry
        zc = z_all_ref[pl.ds(t * _C, _C), :]
        g = jax.lax.dot_general(
            zr, zc, (((1,), (1,)), ((), ())),
            preferred_element_type=jnp.float32)
        buf_ref[smw, :, pl.ds(t * _C, _C)] = g
        mid = 0.5 * (lo + hi)
        thr = 1.0 - 0.5 * mid
        mk = jnp.where(buf_ref[smr] >= thr, 1.0, 0.0)
        cnt = jax.lax.dot_general(
            mk, ones_b, (((1,), (0,)), ((), ())),
            preferred_element_type=jnp.float32)
        ge = cnt >= float(_K + 1)
        return jnp.where(ge, lo, mid), jnp.where(ge, mid, hi)

    lo0 = jnp.zeros((_RB, 1), jnp.float32)
    hi0 = jnp.full((_RB, 1), 4.5, jnp.float32)
    _, hi = jax.lax.fori_loop(0, _NCH, it, (lo0, hi0))
    knn_ref[...] = jnp.sqrt(hi[:, 0])


def _softplus(x):
    return jnp.maximum(x, 0.0) + jnp.log(1.0 + jnp.exp(-jnp.abs(x)))


def _final_body(knn_ref, zw_ref, noise_ref, w_ref, b_ref, out_ref):
    bval = b_ref[0]
    zw = zw_ref[...]
    gw = jax.lax.dot_general(
        noise_ref[...], w_ref[...], (((1,), (0,)), ((), ())),
        preferred_element_type=jnp.float32)
    id_loss = jnp.sum(_softplus(-(zw + bval))) / float(_B)
    iota = jax.lax.broadcasted_iota(jnp.int32, (1, _B), 1)
    v = knn_ref[...]
    ood_sum = jnp.float32(0.0)
    for p in range(_P):
        m = jnp.max(v)
        idx = jnp.min(jnp.where(v == m, iota, _B))
        hit = iota == idx
        zsel = jnp.sum(jnp.where(hit, zw, 0.0))
        ood_sum = ood_sum + _softplus(zsel + bval + _SIGMA * gw[p, 0])
        v = jnp.where(hit, -1.0, v)
    out = _ALPHA * (id_loss + ood_sum / float(_P))
    out_ref[...] = jnp.full((1, 1), out, jnp.float32)


def kernel(embeddings, labels, W, b):
    del labels
    emb = embeddings.astype(jnp.float32)
    w = W.astype(jnp.float32)

    z, zw = pl.pallas_call(
        _prep_body,
        grid=(_B // _RBP,),
        in_specs=[
            pl.BlockSpec((_RBP, _D), lambda i: (i, 0)),
            pl.BlockSpec((_D, 1), lambda i: (0, 0)),
        ],
        out_specs=[
            pl.BlockSpec((_RBP, _D), lambda i: (i, 0)),
            pl.BlockSpec((_RBP,), lambda i: (i,)),
        ],
        out_shape=[
            jax.ShapeDtypeStruct((_B, _D), jnp.float32),
            jax.ShapeDtypeStruct((_B,), jnp.float32),
        ],
    )(emb, w)

    nblk = _B // _RB
    knn = pl.pallas_call(
        _knn_body,
        grid=(nblk + 1,),
        in_specs=[
            pl.BlockSpec((_RB, _D), lambda s: (jnp.minimum(s, nblk - 1), 0)),
            pl.BlockSpec((_B, _D), lambda s: (0, 0)),
        ],
        out_specs=pl.BlockSpec((_RB,), lambda s: (jnp.maximum(s - 1, 0),)),
        out_shape=jax.ShapeDtypeStruct((_B,), jnp.float32),
        scratch_shapes=[pltpu.VMEM((2, _RB, _B), jnp.float32)],
    )(z, z)

    noise = jax.random.normal(jax.random.key(1234), (_P, 1, _D),
                              dtype=jnp.float32).reshape(_P, _D)
    out = pl.pallas_call(
        _final_body,
        in_specs=[
            pl.BlockSpec((1, _B), lambda: (0, 0)),
            pl.BlockSpec((1, _B), lambda: (0, 0)),
            pl.BlockSpec((_P, _D), lambda: (0, 0)),
            pl.BlockSpec((_D, 1), lambda: (0, 0)),
            pl.BlockSpec(memory_space=pltpu.SMEM),
        ],
        out_specs=pl.BlockSpec((1, 1), lambda: (0, 0)),
        out_shape=jax.ShapeDtypeStruct((1, 1), jnp.float32),
    )(knn.reshape(1, _B), zw.reshape(1, _B), noise, w,
      b.astype(jnp.float32))
    return out.reshape(())


# speedup vs baseline: 1.2699x; 1.2699x over previous
import jax
import jax.numpy as jnp
from jax.experimental import pallas as pl
from jax.experimental.pallas import tpu as pltpu

_B = 4096
_D = 1024
_K = 50
_P = 10
_SIGMA = 0.5
_ALPHA = 0.1

_RB = 256
_C = 256
_NCH = _B // _C
_RBP = 512


def _prep_body(emb_ref, w_ref, z_ref, zw_ref):
    x = emb_ref[...]
    ss = jnp.sum(x * x, axis=1, keepdims=True)
    norm = jnp.maximum(jnp.sqrt(ss), 1e-12)
    z = x / norm
    z_ref[...] = z
    zw_ref[...] = jax.lax.dot_general(
        z, w_ref[...], (((1,), (0,)), ((), ())),
        preferred_element_type=jnp.float32)[:, 0]


def _knn_body(z_row_ref, z_all_ref, knn_ref, g_ref):
    g_ref[...] = jax.lax.dot_general(
        z_row_ref[...], z_all_ref[...], (((1,), (1,)), ((), ())),
        preferred_element_type=jnp.float32)
    ones_b = jnp.ones((_B, 1), jnp.float32)
    tgt = float(_K + 1)

    def qpass(_, carry):
        lo, hi = carry
        w4 = 0.25 * (hi - lo)
        gv = g_ref[...]
        cnts = []
        for k in (1, 2, 3):
            thr = 1.0 - 0.5 * (lo + float(k) * w4)
            mk = jnp.where(gv >= thr, 1.0, 0.0)
            cnts.append(jax.lax.dot_general(
                mk, ones_b, (((1,), (0,)), ((), ())),
                preferred_element_type=jnp.float32))
        q = (jnp.where(cnts[0] < tgt, 1.0, 0.0)
             + jnp.where(cnts[1] < tgt, 1.0, 0.0)
             + jnp.where(cnts[2] < tgt, 1.0, 0.0))
        lo2 = lo + q * w4
        return lo2, lo2 + w4

    lo0 = jnp.zeros((_RB, 1), jnp.float32)
    hi0 = jnp.full((_RB, 1), 4.5, jnp.float32)
    _, hi = jax.lax.fori_loop(0, 8, qpass, (lo0, hi0))
    knn_ref[...] = jnp.sqrt(hi[:, 0])


def _softplus(x):
    return jnp.maximum(x, 0.0) + jnp.log(1.0 + jnp.exp(-jnp.abs(x)))


def _final_body(knn_ref, zw_ref, noise_ref, w_ref, b_ref, out_ref):
    bval = b_ref[0]
    zw = zw_ref[...]
    gw = jax.lax.dot_general(
        noise_ref[...], w_ref[...], (((1,), (0,)), ((), ())),
        preferred_element_type=jnp.float32)
    id_loss = jnp.sum(_softplus(-(zw + bval))) / float(_B)
    iota = jax.lax.broadcasted_iota(jnp.int32, (1, _B), 1)
    v = knn_ref[...]
    ood_sum = jnp.float32(0.0)
    for p in range(_P):
        m = jnp.max(v)
        idx = jnp.min(jnp.where(v == m, iota, _B))
        hit = iota == idx
        zsel = jnp.sum(jnp.where(hit, zw, 0.0))
        ood_sum = ood_sum + _softplus(zsel + bval + _SIGMA * gw[p, 0])
        v = jnp.where(hit, -1.0, v)
    out = _ALPHA * (id_loss + ood_sum / float(_P))
    out_ref[...] = jnp.full((1, 1), out, jnp.float32)


def kernel(embeddings, labels, W, b):
    del labels
    emb = embeddings.astype(jnp.float32)
    w = W.astype(jnp.float32)

    z, zw = pl.pallas_call(
        _prep_body,
        grid=(_B // _RBP,),
        in_specs=[
            pl.BlockSpec((_RBP, _D), lambda i: (i, 0)),
            pl.BlockSpec((_D, 1), lambda i: (0, 0)),
        ],
        out_specs=[
            pl.BlockSpec((_RBP, _D), lambda i: (i, 0)),
            pl.BlockSpec((_RBP,), lambda i: (i,)),
        ],
        out_shape=[
            jax.ShapeDtypeStruct((_B, _D), jnp.float32),
            jax.ShapeDtypeStruct((_B,), jnp.float32),
        ],
    )(emb, w)

    nblk = _B // _RB
    knn = pl.pallas_call(
        _knn_body,
        grid=(nblk,),
        in_specs=[
            pl.BlockSpec((_RB, _D), lambda s: (s, 0)),
            pl.BlockSpec((_B, _D), lambda s: (0, 0)),
        ],
        out_specs=pl.BlockSpec((_RB,), lambda s: (s,)),
        out_shape=jax.ShapeDtypeStruct((_B,), jnp.float32),
        scratch_shapes=[pltpu.VMEM((_RB, _B), jnp.float32)],
    )(z, z)

    noise = jax.random.normal(jax.random.key(1234), (_P, 1, _D),
                              dtype=jnp.float32).reshape(_P, _D)
    out = pl.pallas_call(
        _final_body,
        in_specs=[
            pl.BlockSpec((1, _B), lambda: (0, 0)),
            pl.BlockSpec((1, _B), lambda: (0, 0)),
            pl.BlockSpec((_P, _D), lambda: (0, 0)),
            pl.BlockSpec((_D, 1), lambda: (0, 0)),
            pl.BlockSpec(memory_space=pltpu.SMEM),
        ],
        out_specs=pl.BlockSpec((1, 1), lambda: (0, 0)),
        out_shape=jax.ShapeDtypeStruct((1, 1), jnp.float32),
    )(knn.reshape(1, _B), zw.reshape(1, _B), noise, w,
      b.astype(jnp.float32))
    return out.reshape(())


# speedup vs baseline: 1.6800x; 1.3229x over previous
import jax
import jax.numpy as jnp
from jax.experimental import pallas as pl
from jax.experimental.pallas import tpu as pltpu

_B = 4096
_D = 1024
_K = 50
_P = 10
_SIGMA = 0.5
_ALPHA = 0.1

_RB = 256
_C = 256
_NCH = _B // _C
_RBP = 512
_NP = _B // _RBP
_NB = _B // _RB
_NK = _NB // 2


def _softplus(x):
    return jnp.maximum(x, 0.0) + jnp.log(1.0 + jnp.exp(-jnp.abs(x)))


def _count_block(z_ref, src_ref, dst_ref, nxt_blk):
    zr = z_ref[pl.ds(nxt_blk * _RB, _RB), :]

    def it(t, carry):
        lo, hi = carry
        zc = z_ref[pl.ds(t * _C, _C), :]
        g = jax.lax.dot_general(
            zr, zc, (((1,), (1,)), ((), ())),
            preferred_element_type=jnp.float32)
        dst_ref[:, pl.ds(t * _C, _C)] = g
        mid = 0.5 * (lo + hi)
        thr = 1.0 - 0.5 * mid
        cnt = jnp.sum((src_ref[...] >= thr).astype(jnp.float32),
                      axis=1, keepdims=True)
        ge = cnt >= float(_K + 1)
        return jnp.where(ge, lo, mid), jnp.where(ge, mid, hi)

    lo0 = jnp.zeros((_RB, 1), jnp.float32)
    hi0 = jnp.full((_RB, 1), 4.5, jnp.float32)
    _, hi = jax.lax.fori_loop(0, _NCH, it, (lo0, hi0))
    return jnp.sqrt(hi[:, 0])


def _mega_body(emb_ref, w_ref, noise_ref, b_ref, out_ref,
               z_ref, zw_ref, buf_a, buf_b, knn_ref):
    s = pl.program_id(0)

    @pl.when(s < _NP)
    def _prep():
        x = emb_ref[...]
        ss = jnp.sum(x * x, axis=1, keepdims=True)
        norm = jnp.maximum(jnp.sqrt(ss), 1e-12)
        z = x / norm
        z_ref[pl.ds(s * _RBP, _RBP), :] = z
        zw_ref[s] = jax.lax.dot_general(
            z, w_ref[...], (((1,), (0,)), ((), ())),
            preferred_element_type=jnp.float32)[:, 0]

    @pl.when(s == _NP - 1)
    def _prime():
        buf_a[...] = jax.lax.dot_general(
            z_ref[pl.ds(0, _RB), :], z_ref[...], (((1,), (1,)), ((), ())),
            preferred_element_type=jnp.float32)

    @pl.when(jnp.logical_and(s >= _NP, s < _NP + _NK))
    def _knn():
        k2 = s - _NP
        blk = 2 * k2
        knn_ref[blk] = _count_block(z_ref, buf_a, buf_b, blk + 1)
        knn_ref[blk + 1] = _count_block(
            z_ref, buf_b, buf_a, jnp.minimum(blk + 2, _NB - 1))

    @pl.when(s == _NP + _NK)
    def _final():
        bval = b_ref[0]
        zw = zw_ref[...]
        gw = jax.lax.dot_general(
            noise_ref[...], w_ref[...], (((1,), (0,)), ((), ())),
            preferred_element_type=jnp.float32)
        id_loss = jnp.sum(_softplus(-(zw + bval))) / float(_B)
        idx_zw = (jax.lax.broadcasted_iota(jnp.int32, (_NP, _RBP), 0) * _RBP
                  + jax.lax.broadcasted_iota(jnp.int32, (_NP, _RBP), 1))
        idx_kn = (jax.lax.broadcasted_iota(jnp.int32, (_NB, _RB), 0) * _RB
                  + jax.lax.broadcasted_iota(jnp.int32, (_NB, _RB), 1))
        v = knn_ref[...]
        ood_sum = jnp.float32(0.0)
        for p in range(_P):
            m = jnp.max(v)
            gidx = jnp.min(jnp.where(v == m, idx_kn, _B))
            zsel = jnp.sum(jnp.where(idx_zw == gidx, zw, 0.0))
            ood_sum = ood_sum + _softplus(zsel + bval + _SIGMA * gw[p, 0])
            v = jnp.where(idx_kn == gidx, -1.0, v)
        out = _ALPHA * (id_loss + ood_sum / float(_P))
        out_ref[...] = jnp.full((1, 1), out, jnp.float32)


def kernel(embeddings, labels, W, b):
    del labels
    emb = embeddings.astype(jnp.float32)
    w = W.astype(jnp.float32)
    noise = jax.random.normal(jax.random.key(1234), (_P, 1, _D),
                              dtype=jnp.float32).reshape(_P, _D)
    out = pl.pallas_call(
        _mega_body,
        grid=(_NP + _NK + 1,),
        in_specs=[
            pl.BlockSpec((_RBP, _D), lambda s: (jnp.minimum(s, _NP - 1), 0)),
            pl.BlockSpec((_D, 1), lambda s: (0, 0)),
            pl.BlockSpec((_P, _D), lambda s: (0, 0)),
            pl.BlockSpec(memory_space=pltpu.SMEM),
        ],
        out_specs=pl.BlockSpec((1, 1), lambda s: (0, 0)),
        out_shape=jax.ShapeDtypeStruct((1, 1), jnp.float32),
        scratch_shapes=[
            pltpu.VMEM((_B, _D), jnp.float32),
            pltpu.VMEM((_NP, _RBP), jnp.float32),
            pltpu.VMEM((_RB, _B), jnp.float32),
            pltpu.VMEM((_RB, _B), jnp.float32),
            pltpu.VMEM((_NB, _RB), jnp.float32),
        ],
    )(emb, w, noise, b.astype(jnp.float32))
    return out.reshape(())
